# Initial kernel scaffold; baseline (speedup 1.0000x reference)
import jax
import jax.numpy as jnp
from jax.experimental import pallas as pl


def kernel(boxes, scores, labels):
    raise NotImplementedError("write your pallas kernel here")


# speedup vs baseline: 1.2170x; 1.2170x over previous
import functools

import jax
import jax.numpy as jnp
from jax import lax
from jax.experimental import pallas as pl
from jax.experimental.pallas import tpu as pltpu
from jax.experimental.pallas import tpu_sc as plsc

_N = 20000
_NPAD = 20480
_CHUNK = 1280
_VREGS = _CHUNK // 16
_CAP = 304
_BUF = 320
_OUT_ROWS = 304
_THRESH = 0.05


def _sc_body(scores_hbm, labels_hbm, boxes_hbm, out_hbm,
             sc_v, lb_v, cidx_v, cscr_v, clbl_v, cnt_v,
             counts_sh, idx_sh, scr_sh, lbl_sh,
             cnts2_v, bases_v, idx_all, scr_all, lbl_all,
             gidx_v, gq_v, br0, br1, br2, det_v, sem):
    wid = lax.axis_index("s")
    iota = lax.iota(jnp.int32, 16)
    base_row = wid * _CHUNK

    pltpu.sync_copy(scores_hbm.at[pl.ds(base_row, _CHUNK)], sc_v)
    pltpu.sync_copy(labels_hbm.at[pl.ds(base_row, _CHUNK)], lb_v)

    def scan_body(i, off):
        s = sc_v[pl.ds(i * 16, 16)]
        l = lb_v[pl.ds(i * 16, 16)]
        m = s >= _THRESH
        gi = base_row + i * 16 + iota
        cums = plsc.cumsum(jnp.where(m, 1, 0).astype(jnp.int32))
        p = jnp.where(m, jnp.minimum(off + cums - 1, _BUF - 1), _BUF - 1)
        plsc.store_scatter(cidx_v, [p], gi)
        plsc.store_scatter(cscr_v, [p], s)
        plsc.store_scatter(clbl_v, [p], l)
        return off + jnp.max(cums)

    cnt = lax.fori_loop(0, _VREGS, scan_body, jnp.int32(0))

    cnt_v[...] = jnp.full((16,), cnt, jnp.int32)
    pltpu.sync_copy(cnt_v, counts_sh.at[wid])
    pltpu.sync_copy(cidx_v, idx_sh.at[wid])
    pltpu.sync_copy(cscr_v, scr_sh.at[wid])
    pltpu.sync_copy(clbl_v, lbl_sh.at[wid])
    plsc.subcore_barrier()

    @pl.when(wid == 0)
    def _():
        pltpu.sync_copy(counts_sh, cnts2_v)
        c_vec = plsc.load_gather(cnts2_v, [iota, iota * 0])
        inc = plsc.cumsum(c_vec)
        bases = inc - c_vec
        n_keep = jnp.max(inc)
        bases_v[...] = bases

        b_scalars = [bases[k] for k in range(16)]
        for w in range(16):
            bw = b_scalars[w]

            @pl.when(bw < _CAP)
            def _copy():
                pltpu.sync_copy(idx_sh.at[w], idx_all.at[w])
                pltpu.sync_copy(scr_sh.at[w], scr_all.at[w])
                pltpu.sync_copy(lbl_sh.at[w], lbl_all.at[w])

        zero_f = jnp.zeros((16,), jnp.float32)
        zero_i = jnp.zeros((16,), jnp.int32)

        for t in range(_OUT_ROWS // 16):
            jv = t * 16 + iota
            w = jnp.zeros((16,), jnp.int32)
            for k in range(1, 16):
                w = w + jnp.where(jv >= b_scalars[k], 1, 0)
            basew = plsc.load_gather(bases_v, [w])
            local = jv - basew
            g = plsc.load_gather(idx_all, [w, local])
            s = plsc.load_gather(scr_all, [w, local])
            lb = plsc.load_gather(lbl_all, [w, local])
            valid = jv < n_keep
            g0 = jnp.where(valid, g, zero_i)
            s0 = jnp.where(valid, s, zero_f)
            lf = jnp.where(valid, lb.astype(jnp.float32), zero_f)
            plsc.store_scatter(gidx_v, [jnp.full((16,), t // 8, jnp.int32),
                                        jv - (t // 8) * 128],
                               lax.shift_right_logical(g0, 2))
            gq_v[pl.ds(t * 16, 16)] = jnp.bitwise_and(g0, 3)
            plsc.store_scatter(det_v, [jv * 6 + 4], s0)
            plsc.store_scatter(det_v, [jv * 6 + 5], lf)

        c0 = pltpu.async_copy(boxes_hbm.at[gidx_v.at[0]], br0, sem)
        c1 = pltpu.async_copy(boxes_hbm.at[gidx_v.at[1]], br1, sem)
        c2 = pltpu.async_copy(boxes_hbm.at[gidx_v.at[2]], br2, sem)
        c0.wait()
        c1.wait()
        c2.wait()

        br_refs = [br0, br1, br2]
        for t in range(_OUT_ROWS // 16):
            jv = t * 16 + iota
            valid = jv < n_keep
            row_in = jv - (t // 8) * 128
            qcol = gq_v[pl.ds(t * 16, 16)] * 4
            for c in range(4):
                bv = plsc.load_gather(br_refs[t // 8], [row_in, qcol + c])
                bv0 = jnp.where(valid, bv, zero_f)
                plsc.store_scatter(det_v, [jv * 6 + c], bv0)

        pltpu.sync_copy(det_v, out_hbm)


@jax.jit
def kernel(boxes, scores, labels):
    pad = _NPAD - _N
    scores_p = jnp.concatenate(
        [scores, jnp.full((pad,), -1.0, scores.dtype)])
    labels_p = jnp.concatenate(
        [labels.astype(jnp.int32), jnp.zeros((pad,), jnp.int32)])

    mesh = plsc.VectorSubcoreMesh(
        core_axis_name="c", subcore_axis_name="s", num_cores=1)
    flat = pl.kernel(
        _sc_body,
        out_type=jax.ShapeDtypeStruct((_OUT_ROWS * 6,), jnp.float32),
        mesh=mesh,
        compiler_params=pltpu.CompilerParams(
            needs_layout_passes=False, use_tc_tiling_on_sc=False),
        scratch_types=[
            pltpu.VMEM((_CHUNK,), jnp.float32),
            pltpu.VMEM((_CHUNK,), jnp.int32),
            pltpu.VMEM((_BUF,), jnp.int32),
            pltpu.VMEM((_BUF,), jnp.float32),
            pltpu.VMEM((_BUF,), jnp.int32),
            pltpu.VMEM((16,), jnp.int32),
            pltpu.VMEM_SHARED((16, 16), jnp.int32),
            pltpu.VMEM_SHARED((16, _BUF), jnp.int32),
            pltpu.VMEM_SHARED((16, _BUF), jnp.float32),
            pltpu.VMEM_SHARED((16, _BUF), jnp.int32),
            pltpu.VMEM((16, 16), jnp.int32),
            pltpu.VMEM((16,), jnp.int32),
            pltpu.VMEM((16, _BUF), jnp.int32),
            pltpu.VMEM((16, _BUF), jnp.float32),
            pltpu.VMEM((16, _BUF), jnp.int32),
            pltpu.VMEM((3, 128), jnp.int32),
            pltpu.VMEM((_OUT_ROWS,), jnp.int32),
            pltpu.VMEM((128, 16), jnp.float32),
            pltpu.VMEM((128, 16), jnp.float32),
            pltpu.VMEM((128, 16), jnp.float32),
            pltpu.VMEM((_OUT_ROWS * 6,), jnp.float32),
            pltpu.SemaphoreType.DMA,
        ],
    )(scores_p, labels_p, boxes.reshape(_N // 4, 16))
    det = flat.reshape(_OUT_ROWS, 6)[:300]
    return det[None]


# speedup vs baseline: 1.2476x; 1.0251x over previous
import jax
import jax.numpy as jnp
from jax import lax
from jax.experimental import pallas as pl
from jax.experimental.pallas import tpu as pltpu
from jax.experimental.pallas import tpu_sc as plsc

_N = 20000
_CHUNK = 1280
_VREGS = _CHUNK // 16
_LAST = _N - _CHUNK
_CAP = 304
_BUF = 320
_OUT_ROWS = 304
_THRESH = 0.05


def _sc_body(scores_hbm, labels_hbm, boxes_hbm, out_hbm,
             sc_v, lb_v, bx_v,
             cscr_v, clbl_v, cbx0, cbx1, cbx2, cbx3, cnt_v,
             counts_sh, scr_sh, lbl_sh, bx0_sh, bx1_sh, bx2_sh, bx3_sh,
             cnts2_v, bases_v, scr_all, lbl_all,
             bx0_all, bx1_all, bx2_all, bx3_all, det_v):
    wid = lax.axis_index("s")
    iota = lax.iota(jnp.int32, 16)
    chunk_start = wid * _CHUNK
    base_ld = jnp.minimum(chunk_start, _LAST)

    pltpu.sync_copy(scores_hbm.at[pl.ds(base_ld, _CHUNK)], sc_v)
    pltpu.sync_copy(labels_hbm.at[pl.ds(base_ld, _CHUNK)], lb_v)
    pltpu.sync_copy(
        boxes_hbm.at[pl.ds(jnp.minimum(wid * (_CHUNK // 4), _LAST // 4),
                           _CHUNK // 4)], bx_v)

    def count_body(i, acc):
        s = sc_v[pl.ds(i * 16, 16)]
        gi = base_ld + i * 16 + iota
        m = jnp.logical_and(s >= _THRESH, gi >= chunk_start)
        return acc + plsc.all_reduce_population_count(m)

    cnt = lax.fori_loop(0, _VREGS, count_body,
                        jnp.zeros((16,), jnp.int32))[0]

    cbx = [cbx0, cbx1, cbx2, cbx3]

    def comp_cond(carry):
        i, off = carry
        return jnp.logical_and(i < _VREGS, off < _CAP)

    def comp_body(carry):
        i, off = carry
        s = sc_v[pl.ds(i * 16, 16)]
        l = lb_v[pl.ds(i * 16, 16)]
        r = i * 16 + iota
        gi = base_ld + r
        m = jnp.logical_and(s >= _THRESH, gi >= chunk_start)
        cums = plsc.cumsum(jnp.where(m, 1, 0).astype(jnp.int32))
        p = jnp.where(m, jnp.minimum(off + cums - 1, _BUF - 1), _BUF - 1)
        plsc.store_scatter(cscr_v, [p], s)
        plsc.store_scatter(clbl_v, [p], l)
        qr = lax.shift_right_logical(r, 2)
        qc = jnp.bitwise_and(r, 3) * 4
        for c in range(4):
            bc = plsc.load_gather(bx_v, [qr, qc + c])
            plsc.store_scatter(cbx[c], [p], bc)
        return i + 1, off + cums[15]

    lax.while_loop(comp_cond, comp_body, (jnp.int32(0), jnp.int32(0)))

    cnt_v[...] = jnp.full((16,), cnt, jnp.int32)
    pltpu.sync_copy(cnt_v, counts_sh.at[wid])
    pltpu.sync_copy(cscr_v, scr_sh.at[wid])
    pltpu.sync_copy(clbl_v, lbl_sh.at[wid])
    pltpu.sync_copy(cbx0, bx0_sh.at[wid])
    pltpu.sync_copy(cbx1, bx1_sh.at[wid])
    pltpu.sync_copy(cbx2, bx2_sh.at[wid])
    pltpu.sync_copy(cbx3, bx3_sh.at[wid])
    plsc.subcore_barrier()

    @pl.when(wid == 0)
    def _():
        pltpu.sync_copy(counts_sh, cnts2_v)
        c_vec = plsc.load_gather(cnts2_v, [iota, iota * 0])
        inc = plsc.cumsum(c_vec)
        bases = inc - c_vec
        n_keep = inc[15]
        bases_v[...] = bases

        b_scalars = [bases[k] for k in range(16)]
        sh_all = [(scr_sh, scr_all), (lbl_sh, lbl_all),
                  (bx0_sh, bx0_all), (bx1_sh, bx1_all),
                  (bx2_sh, bx2_all), (bx3_sh, bx3_all)]
        for w in range(16):
            bw = b_scalars[w]

            @pl.when(bw < _CAP)
            def _copy():
                for sh, dst in sh_all:
                    pltpu.sync_copy(sh.at[w], dst.at[w])

        zero_f = jnp.zeros((16,), jnp.float32)

        bx_all = [bx0_all, bx1_all, bx2_all, bx3_all]
        for t in range(_OUT_ROWS // 16):
            jv = t * 16 + iota
            w = jnp.zeros((16,), jnp.int32)
            for k in range(1, 16):
                w = w + jnp.where(jv >= b_scalars[k], 1, 0)
            basew = plsc.load_gather(bases_v, [w])
            local = jv - basew
            valid = jv < n_keep
            for c in range(4):
                bv = plsc.load_gather(bx_all[c], [w, local])
                plsc.store_scatter(det_v, [jv * 6 + c],
                                   jnp.where(valid, bv, zero_f))
            s = plsc.load_gather(scr_all, [w, local])
            lb = plsc.load_gather(lbl_all, [w, local])
            plsc.store_scatter(det_v, [jv * 6 + 4],
                               jnp.where(valid, s, zero_f))
            plsc.store_scatter(det_v, [jv * 6 + 5],
                               jnp.where(valid, lb.astype(jnp.float32),
                                         zero_f))

        pltpu.sync_copy(det_v, out_hbm)


@jax.jit
def kernel(boxes, scores, labels):
    mesh = plsc.VectorSubcoreMesh(
        core_axis_name="c", subcore_axis_name="s", num_cores=1)
    flat = pl.kernel(
        _sc_body,
        out_type=jax.ShapeDtypeStruct((_OUT_ROWS * 6,), jnp.float32),
        mesh=mesh,
        compiler_params=pltpu.CompilerParams(
            needs_layout_passes=False, use_tc_tiling_on_sc=False),
        scratch_types=[
            pltpu.VMEM((_CHUNK,), jnp.float32),
            pltpu.VMEM((_CHUNK,), jnp.int32),
            pltpu.VMEM((_CHUNK // 4, 16), jnp.float32),
            pltpu.VMEM((_BUF,), jnp.float32),
            pltpu.VMEM((_BUF,), jnp.int32),
            pltpu.VMEM((_BUF,), jnp.float32),
            pltpu.VMEM((_BUF,), jnp.float32),
            pltpu.VMEM((_BUF,), jnp.float32),
            pltpu.VMEM((_BUF,), jnp.float32),
            pltpu.VMEM((16,), jnp.int32),
            pltpu.VMEM_SHARED((16, 16), jnp.int32),
            pltpu.VMEM_SHARED((16, _BUF), jnp.float32),
            pltpu.VMEM_SHARED((16, _BUF), jnp.int32),
            pltpu.VMEM_SHARED((16, _BUF), jnp.float32),
            pltpu.VMEM_SHARED((16, _BUF), jnp.float32),
            pltpu.VMEM_SHARED((16, _BUF), jnp.float32),
            pltpu.VMEM_SHARED((16, _BUF), jnp.float32),
            pltpu.VMEM((16, 16), jnp.int32),
            pltpu.VMEM((16,), jnp.int32),
            pltpu.VMEM((16, _BUF), jnp.float32),
            pltpu.VMEM((16, _BUF), jnp.int32),
            pltpu.VMEM((16, _BUF), jnp.float32),
            pltpu.VMEM((16, _BUF), jnp.float32),
            pltpu.VMEM((16, _BUF), jnp.float32),
            pltpu.VMEM((16, _BUF), jnp.float32),
            pltpu.VMEM((_OUT_ROWS * 6,), jnp.float32),
        ],
    )(scores, labels.astype(jnp.int32), boxes.reshape(_N // 4, 16))
    return flat.reshape(_OUT_ROWS, 6)[:300][None]


# speedup vs baseline: 1.2691x; 1.0172x over previous
import jax
import jax.numpy as jnp
from jax import lax
from jax.experimental import pallas as pl
from jax.experimental.pallas import tpu as pltpu
from jax.experimental.pallas import tpu_sc as plsc

_N = 20000
_CHUNK = 1280
_VREGS = _CHUNK // 16
_LAST = _N - _CHUNK
_CAP = 304
_BUF = 320
_OUT_ROWS = 304
_THRESH = 0.05


def _sc_body(scores_hbm, labels_hbm, boxes_hbm, out_hbm,
             sc_v, lb_v, bx_v,
             cscr_v, clbl_v, cbx0, cbx1, cbx2, cbx3, cnt_v,
             counts_sh, scr_sh, lbl_sh, bx0_sh, bx1_sh, bx2_sh, bx3_sh,
             cnts2_v, bases_v, scr_all, lbl_all,
             bx0_all, bx1_all, bx2_all, bx3_all, det_v,
             sem0, sem1, sem2):
    wid = lax.axis_index("s")
    iota = lax.iota(jnp.int32, 16)
    chunk_start = wid * _CHUNK
    base_ld = jnp.minimum(chunk_start, _LAST)

    ld0 = pltpu.async_copy(scores_hbm.at[pl.ds(base_ld, _CHUNK)], sc_v, sem0)
    ld1 = pltpu.async_copy(labels_hbm.at[pl.ds(base_ld, _CHUNK)], lb_v, sem1)
    ld2 = pltpu.async_copy(
        boxes_hbm.at[pl.ds(jnp.minimum(wid * (_CHUNK // 4), _LAST // 4),
                           _CHUNK // 4)], bx_v, sem2)
    ld0.wait()
    ld1.wait()
    ld2.wait()

    def count_body(i, acc):
        s = sc_v[pl.ds(i * 16, 16)]
        gi = base_ld + i * 16 + iota
        m = jnp.logical_and(s >= _THRESH, gi >= chunk_start)
        return acc + plsc.all_reduce_population_count(m)

    cnt = lax.fori_loop(0, _VREGS, count_body,
                        jnp.zeros((16,), jnp.int32))[0]

    cbx = [cbx0, cbx1, cbx2, cbx3]

    def comp_cond(carry):
        i, off = carry
        return jnp.logical_and(i < _VREGS, off < _CAP)

    def comp_body(carry):
        i, off = carry
        s = sc_v[pl.ds(i * 16, 16)]
        l = lb_v[pl.ds(i * 16, 16)]
        r = i * 16 + iota
        gi = base_ld + r
        m = jnp.logical_and(s >= _THRESH, gi >= chunk_start)
        cums = plsc.cumsum(jnp.where(m, 1, 0).astype(jnp.int32))
        p = jnp.where(m, jnp.minimum(off + cums - 1, _BUF - 1), _BUF - 1)
        plsc.store_scatter(cscr_v, [p], s)
        plsc.store_scatter(clbl_v, [p], l)
        qr = lax.shift_right_logical(r, 2)
        qc = jnp.bitwise_and(r, 3) * 4
        for c in range(4):
            bc = plsc.load_gather(bx_v, [qr, qc + c])
            plsc.store_scatter(cbx[c], [p], bc)
        return i + 1, off + cums[15]

    lax.while_loop(comp_cond, comp_body, (jnp.int32(0), jnp.int32(0)))

    cnt_v[...] = jnp.full((16,), cnt, jnp.int32)
    pltpu.sync_copy(cnt_v, counts_sh.at[wid])
    pltpu.sync_copy(cscr_v, scr_sh.at[wid])
    pltpu.sync_copy(clbl_v, lbl_sh.at[wid])
    pltpu.sync_copy(cbx0, bx0_sh.at[wid])
    pltpu.sync_copy(cbx1, bx1_sh.at[wid])
    pltpu.sync_copy(cbx2, bx2_sh.at[wid])
    pltpu.sync_copy(cbx3, bx3_sh.at[wid])
    plsc.subcore_barrier()

    @pl.when(wid == 0)
    def _():
        pltpu.sync_copy(counts_sh, cnts2_v)
        c_vec = plsc.load_gather(cnts2_v, [iota, iota * 0])
        inc = plsc.cumsum(c_vec)
        bases = inc - c_vec
        n_keep = inc[15]
        bases_v[...] = bases

        sh_all = [(scr_sh, scr_all), (lbl_sh, lbl_all),
                  (bx0_sh, bx0_all), (bx1_sh, bx1_all),
                  (bx2_sh, bx2_all), (bx3_sh, bx3_all)]

        def copy_body(w, _):
            bw = plsc.load_gather(bases_v, [jnp.full((16,), w, jnp.int32)])

            @pl.when(bw[0] < _CAP)
            def _copy():
                for sh, dst in sh_all:
                    pltpu.sync_copy(sh.at[w], dst.at[w])

            return 0

        lax.fori_loop(0, 16, copy_body, 0)

        zero_f = jnp.zeros((16,), jnp.float32)
        bx_all = [bx0_all, bx1_all, bx2_all, bx3_all]

        def slot_body(t, _):
            jv = t * 16 + iota

            def search_body(k, wacc):
                bk = plsc.load_gather(bases_v, [jnp.full((16,), k, jnp.int32)])
                return wacc + jnp.where(jv >= bk, 1, 0)

            w = lax.fori_loop(1, 16, search_body, jnp.zeros((16,), jnp.int32))
            basew = plsc.load_gather(bases_v, [w])
            local = jv - basew
            valid = jv < n_keep
            for c in range(4):
                bv = plsc.load_gather(bx_all[c], [w, local])
                plsc.store_scatter(det_v, [jv * 6 + c],
                                   jnp.where(valid, bv, zero_f))
            s = plsc.load_gather(scr_all, [w, local])
            lb = plsc.load_gather(lbl_all, [w, local])
            plsc.store_scatter(det_v, [jv * 6 + 4],
                               jnp.where(valid, s, zero_f))
            plsc.store_scatter(det_v, [jv * 6 + 5],
                               jnp.where(valid, lb.astype(jnp.float32),
                                         zero_f))
            return 0

        lax.fori_loop(0, _OUT_ROWS // 16, slot_body, 0)

        pltpu.sync_copy(det_v, out_hbm)


@jax.jit
def kernel(boxes, scores, labels):
    mesh = plsc.VectorSubcoreMesh(
        core_axis_name="c", subcore_axis_name="s", num_cores=1)
    flat = pl.kernel(
        _sc_body,
        out_type=jax.ShapeDtypeStruct((_OUT_ROWS * 6,), jnp.float32),
        mesh=mesh,
        compiler_params=pltpu.CompilerParams(
            needs_layout_passes=False, use_tc_tiling_on_sc=False),
        scratch_types=[
            pltpu.VMEM((_CHUNK,), jnp.float32),
            pltpu.VMEM((_CHUNK,), jnp.int32),
            pltpu.VMEM((_CHUNK // 4, 16), jnp.float32),
            pltpu.VMEM((_BUF,), jnp.float32),
            pltpu.VMEM((_BUF,), jnp.int32),
            pltpu.VMEM((_BUF,), jnp.float32),
            pltpu.VMEM((_BUF,), jnp.float32),
            pltpu.VMEM((_BUF,), jnp.float32),
            pltpu.VMEM((_BUF,), jnp.float32),
            pltpu.VMEM((16,), jnp.int32),
            pltpu.VMEM_SHARED((16, 16), jnp.int32),
            pltpu.VMEM_SHARED((16, _BUF), jnp.float32),
            pltpu.VMEM_SHARED((16, _BUF), jnp.int32),
            pltpu.VMEM_SHARED((16, _BUF), jnp.float32),
            pltpu.VMEM_SHARED((16, _BUF), jnp.float32),
            pltpu.VMEM_SHARED((16, _BUF), jnp.float32),
            pltpu.VMEM_SHARED((16, _BUF), jnp.float32),
            pltpu.VMEM((16, 16), jnp.int32),
            pltpu.VMEM((16,), jnp.int32),
            pltpu.VMEM((16, _BUF), jnp.float32),
            pltpu.VMEM((16, _BUF), jnp.int32),
            pltpu.VMEM((16, _BUF), jnp.float32),
            pltpu.VMEM((16, _BUF), jnp.float32),
            pltpu.VMEM((16, _BUF), jnp.float32),
            pltpu.VMEM((16, _BUF), jnp.float32),
            pltpu.VMEM((_OUT_ROWS * 6,), jnp.float32),
            pltpu.SemaphoreType.DMA,
            pltpu.SemaphoreType.DMA,
            pltpu.SemaphoreType.DMA,
        ],
    )(scores, labels.astype(jnp.int32), boxes.reshape(_N // 4, 16))
    return flat.reshape(_OUT_ROWS, 6)[:300][None]


# speedup vs baseline: 1.2752x; 1.0048x over previous
import jax
import jax.numpy as jnp
from jax import lax
from jax.experimental import pallas as pl
from jax.experimental.pallas import tpu as pltpu
from jax.experimental.pallas import tpu_sc as plsc

_N = 20000
_CHUNK = 1280
_VREGS = _CHUNK // 16
_LAST = _N - _CHUNK
_CAP = 304
_BUF = 320
_OUT_ROWS = 304
_THRESH = 0.05


def _sc_body(scores_hbm, labels_hbm, boxes_hbm, out_hbm,
             sc_v, lb_v, bx_v,
             cscr_v, clbl_v, cbx0, cbx1, cbx2, cbx3, cnt_v,
             counts_sh, scr_sh, lbl_sh, bx0_sh, bx1_sh, bx2_sh, bx3_sh,
             cnts2_v, bases_v, scr_all, lbl_all,
             bx0_all, bx1_all, bx2_all, bx3_all, det_v,
             sem0, sem1, sem2):
    wid = lax.axis_index("s")
    iota = lax.iota(jnp.int32, 16)
    chunk_start = wid * _CHUNK
    base_ld = jnp.minimum(chunk_start, _LAST)

    ld0 = pltpu.async_copy(scores_hbm.at[pl.ds(base_ld, _CHUNK)], sc_v, sem0)
    ld1 = pltpu.async_copy(labels_hbm.at[pl.ds(base_ld, _CHUNK)], lb_v, sem1)
    ld2 = pltpu.async_copy(
        boxes_hbm.at[pl.ds(jnp.minimum(wid * (_CHUNK // 4), _LAST // 4),
                           _CHUNK // 4)], bx_v, sem2)
    ld0.wait()
    ld1.wait()
    ld2.wait()

    def count_body(i, acc):
        s = sc_v[pl.ds(i * 16, 16)]
        gi = base_ld + i * 16 + iota
        m = jnp.logical_and(s >= _THRESH, gi >= chunk_start)
        return acc + plsc.all_reduce_population_count(m)

    cnt = lax.fori_loop(0, _VREGS, count_body,
                        jnp.zeros((16,), jnp.int32))[0]

    cbx = [cbx0, cbx1, cbx2, cbx3]

    def comp_cond(carry):
        i, off = carry
        return jnp.logical_and(i < _VREGS, off < _CAP)

    def comp_body(carry):
        i, off = carry
        s = sc_v[pl.ds(i * 16, 16)]
        l = lb_v[pl.ds(i * 16, 16)]
        r = i * 16 + iota
        gi = base_ld + r
        m = jnp.logical_and(s >= _THRESH, gi >= chunk_start)
        cums = plsc.cumsum(jnp.where(m, 1, 0).astype(jnp.int32))
        p = jnp.where(m, jnp.minimum(off + cums - 1, _BUF - 1), _BUF - 1)
        plsc.store_scatter(cscr_v, [p], s)
        plsc.store_scatter(clbl_v, [p], l)
        qr = lax.shift_right_logical(r, 2)
        qc = jnp.bitwise_and(r, 3) * 4
        for c in range(4):
            bc = plsc.load_gather(bx_v, [qr, qc + c])
            plsc.store_scatter(cbx[c], [p], bc)
        return i + 1, off + cums[15]

    lax.while_loop(comp_cond, comp_body, (jnp.int32(0), jnp.int32(0)))

    cnt_v[...] = jnp.full((16,), cnt, jnp.int32)
    pltpu.sync_copy(cnt_v, counts_sh.at[wid])
    pltpu.sync_copy(cscr_v, scr_sh.at[wid])
    pltpu.sync_copy(clbl_v, lbl_sh.at[wid])
    pltpu.sync_copy(cbx0, bx0_sh.at[wid])
    pltpu.sync_copy(cbx1, bx1_sh.at[wid])
    pltpu.sync_copy(cbx2, bx2_sh.at[wid])
    pltpu.sync_copy(cbx3, bx3_sh.at[wid])
    plsc.subcore_barrier()

    pltpu.sync_copy(counts_sh, cnts2_v)
    c_vec = plsc.load_gather(cnts2_v, [iota, iota * 0])
    inc = plsc.cumsum(c_vec)
    bases = inc - c_vec
    n_keep = inc[15]
    bases_v[...] = bases

    b_scalars = [bases[k] for k in range(16)]
    e_scalars = [inc[k] for k in range(16)]
    sh_all = [(scr_sh, scr_all), (lbl_sh, lbl_all),
              (bx0_sh, bx0_all), (bx1_sh, bx1_all),
              (bx2_sh, bx2_all), (bx3_sh, bx3_all)]
    zero_f = jnp.zeros((16,), jnp.float32)
    bx_all = [bx0_all, bx1_all, bx2_all, bx3_all]

    def do_vreg(t):
        lo = t * 16
        for w in range(16):
            need = jnp.logical_and(b_scalars[w] <= lo + 15,
                                   e_scalars[w] > lo)

            @pl.when(need)
            def _copy():
                for sh, dst in sh_all:
                    pltpu.sync_copy(sh.at[w], dst.at[w])

        jv = lo + iota
        w = jnp.zeros((16,), jnp.int32)
        for k in range(1, 16):
            w = w + jnp.where(jv >= b_scalars[k], 1, 0)
        basew = plsc.load_gather(bases_v, [w])
        local = jv - basew
        valid = jv < n_keep
        pos = iota * 6
        for c in range(4):
            bv = plsc.load_gather(bx_all[c], [w, local])
            plsc.store_scatter(det_v, [pos + c], jnp.where(valid, bv, zero_f))
        s = plsc.load_gather(scr_all, [w, local])
        lb = plsc.load_gather(lbl_all, [w, local])
        plsc.store_scatter(det_v, [pos + 4], jnp.where(valid, s, zero_f))
        plsc.store_scatter(det_v, [pos + 5],
                           jnp.where(valid, lb.astype(jnp.float32), zero_f))
        pltpu.sync_copy(det_v, out_hbm.at[pl.ds(t * 96, 96)])

    do_vreg(wid)

    @pl.when(wid < (_OUT_ROWS // 16) - 16)
    def _tail():
        do_vreg(wid + 16)


@jax.jit
def kernel(boxes, scores, labels):
    mesh = plsc.VectorSubcoreMesh(
        core_axis_name="c", subcore_axis_name="s", num_cores=1)
    flat = pl.kernel(
        _sc_body,
        out_type=jax.ShapeDtypeStruct((_OUT_ROWS * 6,), jnp.float32),
        mesh=mesh,
        compiler_params=pltpu.CompilerParams(
            needs_layout_passes=False, use_tc_tiling_on_sc=False),
        scratch_types=[
            pltpu.VMEM((_CHUNK,), jnp.float32),
            pltpu.VMEM((_CHUNK,), jnp.int32),
            pltpu.VMEM((_CHUNK // 4, 16), jnp.float32),
            pltpu.VMEM((_BUF,), jnp.float32),
            pltpu.VMEM((_BUF,), jnp.int32),
            pltpu.VMEM((_BUF,), jnp.float32),
            pltpu.VMEM((_BUF,), jnp.float32),
            pltpu.VMEM((_BUF,), jnp.float32),
            pltpu.VMEM((_BUF,), jnp.float32),
            pltpu.VMEM((16,), jnp.int32),
            pltpu.VMEM_SHARED((16, 16), jnp.int32),
            pltpu.VMEM_SHARED((16, _BUF), jnp.float32),
            pltpu.VMEM_SHARED((16, _BUF), jnp.int32),
            pltpu.VMEM_SHARED((16, _BUF), jnp.float32),
            pltpu.VMEM_SHARED((16, _BUF), jnp.float32),
            pltpu.VMEM_SHARED((16, _BUF), jnp.float32),
            pltpu.VMEM_SHARED((16, _BUF), jnp.float32),
            pltpu.VMEM((16, 16), jnp.int32),
            pltpu.VMEM((16,), jnp.int32),
            pltpu.VMEM((16, _BUF), jnp.float32),
            pltpu.VMEM((16, _BUF), jnp.int32),
            pltpu.VMEM((16, _BUF), jnp.float32),
            pltpu.VMEM((16, _BUF), jnp.float32),
            pltpu.VMEM((16, _BUF), jnp.float32),
            pltpu.VMEM((16, _BUF), jnp.float32),
            pltpu.VMEM((96,), jnp.float32),
            pltpu.SemaphoreType.DMA,
            pltpu.SemaphoreType.DMA,
            pltpu.SemaphoreType.DMA,
        ],
    )(scores, labels.astype(jnp.int32), boxes.reshape(_N // 4, 16))
    return flat.reshape(_OUT_ROWS, 6)[:300][None]


# speedup vs baseline: 1.2828x; 1.0059x over previous
import jax
import jax.numpy as jnp
from jax import lax
from jax.experimental import pallas as pl
from jax.experimental.pallas import tpu as pltpu
from jax.experimental.pallas import tpu_sc as plsc

_N = 20000
_CHUNK = 1280
_VREGS = _CHUNK // 16
_LAST = _N - _CHUNK
_CAP = 304
_BUF = 320
_OUT_ROWS = 304
_THRESH = 0.05


def _sc_body(scores_hbm, labels_hbm, boxes_hbm, out_hbm,
             sc_v, lb_v, bx_v,
             cscr_v, clbl_v, cbx0, cbx1, cbx2, cbx3, cnt_v,
             counts_sh, scr_sh, lbl_sh, bx0_sh, bx1_sh, bx2_sh, bx3_sh,
             cnts2_v, bases_v, scr_all, lbl_all,
             bx0_all, bx1_all, bx2_all, bx3_all, det_v,
             sem0, sem1, sem2):
    wid = lax.axis_index("s")
    iota = lax.iota(jnp.int32, 16)
    chunk_start = wid * _CHUNK
    base_ld = jnp.minimum(chunk_start, _LAST)

    ld0 = pltpu.async_copy(scores_hbm.at[pl.ds(base_ld, _CHUNK)], sc_v, sem0)
    ld1 = pltpu.async_copy(labels_hbm.at[pl.ds(base_ld, _CHUNK)], lb_v, sem1)
    ld2 = pltpu.async_copy(
        boxes_hbm.at[pl.ds(jnp.minimum(wid * (_CHUNK // 4), _LAST // 4),
                           _CHUNK // 4)], bx_v, sem2)
    ld0.wait()
    ld1.wait()
    ld2.wait()

    @plsc.parallel_loop(0, _VREGS, unroll=8,
                        carry=jnp.zeros((16,), jnp.int32))
    def count_loop(i, acc):
        s = sc_v[pl.ds(i * 16, 16)]
        gi = base_ld + i * 16 + iota
        m = jnp.logical_and(s >= _THRESH, gi >= chunk_start)
        return acc + plsc.all_reduce_population_count(m)

    cnt = count_loop[0]

    cbx = [cbx0, cbx1, cbx2, cbx3]

    def comp_cond(carry):
        i, off = carry
        return jnp.logical_and(i < _VREGS, off < _CAP)

    def comp_body(carry):
        i, off = carry
        s = sc_v[pl.ds(i * 16, 16)]
        l = lb_v[pl.ds(i * 16, 16)]
        r = i * 16 + iota
        gi = base_ld + r
        m = jnp.logical_and(s >= _THRESH, gi >= chunk_start)
        cums = plsc.cumsum(jnp.where(m, 1, 0).astype(jnp.int32))
        p = jnp.where(m, jnp.minimum(off + cums - 1, _BUF - 1), _BUF - 1)
        plsc.store_scatter(cscr_v, [p], s)
        plsc.store_scatter(clbl_v, [p], l)
        qr = lax.shift_right_logical(r, 2)
        qc = jnp.bitwise_and(r, 3) * 4
        for c in range(4):
            bc = plsc.load_gather(bx_v, [qr, qc + c])
            plsc.store_scatter(cbx[c], [p], bc)
        return i + 1, off + cums[15]

    lax.while_loop(comp_cond, comp_body, (jnp.int32(0), jnp.int32(0)))

    cnt_v[...] = jnp.full((16,), cnt, jnp.int32)
    pltpu.sync_copy(cnt_v, counts_sh.at[wid])
    pltpu.sync_copy(cscr_v, scr_sh.at[wid])
    pltpu.sync_copy(clbl_v, lbl_sh.at[wid])
    pltpu.sync_copy(cbx0, bx0_sh.at[wid])
    pltpu.sync_copy(cbx1, bx1_sh.at[wid])
    pltpu.sync_copy(cbx2, bx2_sh.at[wid])
    pltpu.sync_copy(cbx3, bx3_sh.at[wid])
    plsc.subcore_barrier()

    pltpu.sync_copy(counts_sh, cnts2_v)
    c_vec = plsc.load_gather(cnts2_v, [iota, iota * 0])
    inc = plsc.cumsum(c_vec)
    bases = inc - c_vec
    n_keep = inc[15]
    bases_v[...] = bases

    b_scalars = [bases[k] for k in range(16)]
    e_scalars = [inc[k] for k in range(16)]
    sh_all = [(scr_sh, scr_all), (lbl_sh, lbl_all),
              (bx0_sh, bx0_all), (bx1_sh, bx1_all),
              (bx2_sh, bx2_all), (bx3_sh, bx3_all)]
    zero_f = jnp.zeros((16,), jnp.float32)
    bx_all = [bx0_all, bx1_all, bx2_all, bx3_all]

    def do_vreg(t):
        lo = t * 16
        for w in range(16):
            need = jnp.logical_and(b_scalars[w] <= lo + 15,
                                   e_scalars[w] > lo)

            @pl.when(need)
            def _copy():
                for sh, dst in sh_all:
                    pltpu.sync_copy(sh.at[w], dst.at[w])

        jv = lo + iota
        w = jnp.zeros((16,), jnp.int32)
        for k in range(1, 16):
            w = w + jnp.where(jv >= b_scalars[k], 1, 0)
        basew = plsc.load_gather(bases_v, [w])
        local = jv - basew
        valid = jv < n_keep
        pos = iota * 6
        for c in range(4):
            bv = plsc.load_gather(bx_all[c], [w, local])
            plsc.store_scatter(det_v, [pos + c], jnp.where(valid, bv, zero_f))
        s = plsc.load_gather(scr_all, [w, local])
        lb = plsc.load_gather(lbl_all, [w, local])
        plsc.store_scatter(det_v, [pos + 4], jnp.where(valid, s, zero_f))
        plsc.store_scatter(det_v, [pos + 5],
                           jnp.where(valid, lb.astype(jnp.float32), zero_f))
        pltpu.sync_copy(det_v, out_hbm.at[pl.ds(t * 96, 96)])

    do_vreg(wid)

    @pl.when(wid < (_OUT_ROWS // 16) - 16)
    def _tail():
        do_vreg(wid + 16)


@jax.jit
def kernel(boxes, scores, labels):
    mesh = plsc.VectorSubcoreMesh(
        core_axis_name="c", subcore_axis_name="s", num_cores=1)
    flat = pl.kernel(
        _sc_body,
        out_type=jax.ShapeDtypeStruct((_OUT_ROWS * 6,), jnp.float32),
        mesh=mesh,
        compiler_params=pltpu.CompilerParams(
            needs_layout_passes=False, use_tc_tiling_on_sc=False),
        scratch_types=[
            pltpu.VMEM((_CHUNK,), jnp.float32),
            pltpu.VMEM((_CHUNK,), jnp.int32),
            pltpu.VMEM((_CHUNK // 4, 16), jnp.float32),
            pltpu.VMEM((_BUF,), jnp.float32),
            pltpu.VMEM((_BUF,), jnp.int32),
            pltpu.VMEM((_BUF,), jnp.float32),
            pltpu.VMEM((_BUF,), jnp.float32),
            pltpu.VMEM((_BUF,), jnp.float32),
            pltpu.VMEM((_BUF,), jnp.float32),
            pltpu.VMEM((16,), jnp.int32),
            pltpu.VMEM_SHARED((16, 16), jnp.int32),
            pltpu.VMEM_SHARED((16, _BUF), jnp.float32),
            pltpu.VMEM_SHARED((16, _BUF), jnp.int32),
            pltpu.VMEM_SHARED((16, _BUF), jnp.float32),
            pltpu.VMEM_SHARED((16, _BUF), jnp.float32),
            pltpu.VMEM_SHARED((16, _BUF), jnp.float32),
            pltpu.VMEM_SHARED((16, _BUF), jnp.float32),
            pltpu.VMEM((16, 16), jnp.int32),
            pltpu.VMEM((16,), jnp.int32),
            pltpu.VMEM((16, _BUF), jnp.float32),
            pltpu.VMEM((16, _BUF), jnp.int32),
            pltpu.VMEM((16, _BUF), jnp.float32),
            pltpu.VMEM((16, _BUF), jnp.float32),
            pltpu.VMEM((16, _BUF), jnp.float32),
            pltpu.VMEM((16, _BUF), jnp.float32),
            pltpu.VMEM((96,), jnp.float32),
            pltpu.SemaphoreType.DMA,
            pltpu.SemaphoreType.DMA,
            pltpu.SemaphoreType.DMA,
        ],
    )(scores, labels.astype(jnp.int32), boxes.reshape(_N // 4, 16))
    return flat.reshape(_OUT_ROWS, 6)[:300][None]


# speedup vs baseline: 1.3282x; 1.0354x over previous
import jax
import jax.numpy as jnp
from jax import lax
from jax.experimental import pallas as pl
from jax.experimental.pallas import tpu as pltpu
from jax.experimental.pallas import tpu_sc as plsc

_N = 20000
_CHUNK = 1280
_VREGS = _CHUNK // 16
_LAST = _N - _CHUNK
_CAP = 304
_BUF = 320
_OUT_ROWS = 304
_THRESH = 0.05


def _sc_body(scores_hbm, labels_hbm, boxes_hbm, out_hbm,
             sc_v, cidx_v, cscr_v, cnt_v,
             counts_sh, idx_sh, scr_sh,
             cnts2_v, bases_v, idx_all, scr_all,
             gidxb_v, gidxl_v, brow_v, lrow_v, det_v, semb, seml):
    wid = lax.axis_index("s")
    iota = lax.iota(jnp.int32, 16)
    chunk_start = wid * _CHUNK
    base_ld = jnp.minimum(chunk_start, _LAST)

    pltpu.sync_copy(scores_hbm.at[pl.ds(base_ld, _CHUNK)], sc_v)

    @plsc.parallel_loop(0, _VREGS, unroll=8,
                        carry=jnp.zeros((16,), jnp.int32))
    def count_loop(i, acc):
        s = sc_v[pl.ds(i * 16, 16)]
        gi = base_ld + i * 16 + iota
        m = jnp.logical_and(s >= _THRESH, gi >= chunk_start)
        return acc + plsc.all_reduce_population_count(m)

    cnt = count_loop[0]

    def comp_cond(carry):
        i, off = carry
        return jnp.logical_and(i < _VREGS, off < _CAP)

    def comp_body(carry):
        i, off = carry
        s = sc_v[pl.ds(i * 16, 16)]
        gi = base_ld + i * 16 + iota
        m = jnp.logical_and(s >= _THRESH, gi >= chunk_start)
        cums = plsc.cumsum(jnp.where(m, 1, 0).astype(jnp.int32))
        p = jnp.where(m, jnp.minimum(off + cums - 1, _BUF - 1), _BUF - 1)
        plsc.store_scatter(cidx_v, [p], gi)
        plsc.store_scatter(cscr_v, [p], s)
        return i + 1, off + cums[15]

    lax.while_loop(comp_cond, comp_body, (jnp.int32(0), jnp.int32(0)))

    cnt_v[...] = jnp.full((16,), cnt, jnp.int32)
    pltpu.sync_copy(cnt_v, counts_sh.at[wid])
    pltpu.sync_copy(cidx_v, idx_sh.at[wid])
    pltpu.sync_copy(cscr_v, scr_sh.at[wid])
    plsc.subcore_barrier()

    pltpu.sync_copy(counts_sh, cnts2_v)
    c_vec = plsc.load_gather(cnts2_v, [iota, iota * 0])
    inc = plsc.cumsum(c_vec)
    bases = inc - c_vec
    n_keep = inc[15]
    bases_v[...] = bases

    b_scalars = [bases[k] for k in range(16)]
    e_scalars = [inc[k] for k in range(16)]
    zero_f = jnp.zeros((16,), jnp.float32)

    def do_vreg(t):
        lo = t * 16
        for w in range(16):
            need = jnp.logical_and(b_scalars[w] <= lo + 15,
                                   e_scalars[w] > lo)

            @pl.when(need)
            def _copy():
                pltpu.sync_copy(idx_sh.at[w], idx_all.at[w])
                pltpu.sync_copy(scr_sh.at[w], scr_all.at[w])

        jv = lo + iota
        w = jnp.zeros((16,), jnp.int32)
        for k in range(1, 16):
            w = w + jnp.where(jv >= b_scalars[k], 1, 0)
        basew = plsc.load_gather(bases_v, [w])
        local = jv - basew
        valid = jv < n_keep
        g = plsc.load_gather(idx_all, [w, local])
        s = plsc.load_gather(scr_all, [w, local])
        g0 = jnp.where(valid, g, jnp.zeros((16,), jnp.int32))

        gidxb_v[...] = lax.shift_right_logical(g0, 2)
        gidxl_v[...] = lax.shift_right_logical(g0, 4)
        cb = pltpu.async_copy(boxes_hbm.at[gidxb_v], brow_v, semb)
        cl = pltpu.async_copy(labels_hbm.at[gidxl_v], lrow_v, seml)
        cb.wait()
        cl.wait()

        pos = iota * 6
        qc = jnp.bitwise_and(g0, 3) * 4
        for c in range(4):
            bv = plsc.load_gather(brow_v, [iota, qc + c])
            plsc.store_scatter(det_v, [pos + c], jnp.where(valid, bv, zero_f))
        plsc.store_scatter(det_v, [pos + 4], jnp.where(valid, s, zero_f))
        lb = plsc.load_gather(lrow_v, [iota, jnp.bitwise_and(g0, 15)])
        plsc.store_scatter(det_v, [pos + 5],
                           jnp.where(valid, lb.astype(jnp.float32), zero_f))
        pltpu.sync_copy(det_v, out_hbm.at[pl.ds(t * 96, 96)])

    do_vreg(wid)

    @pl.when(wid < (_OUT_ROWS // 16) - 16)
    def _tail():
        do_vreg(wid + 16)


@jax.jit
def kernel(boxes, scores, labels):
    mesh = plsc.VectorSubcoreMesh(
        core_axis_name="c", subcore_axis_name="s", num_cores=1)
    flat = pl.kernel(
        _sc_body,
        out_type=jax.ShapeDtypeStruct((_OUT_ROWS * 6,), jnp.float32),
        mesh=mesh,
        compiler_params=pltpu.CompilerParams(
            needs_layout_passes=False, use_tc_tiling_on_sc=False),
        scratch_types=[
            pltpu.VMEM((_CHUNK,), jnp.float32),
            pltpu.VMEM((_BUF,), jnp.int32),
            pltpu.VMEM((_BUF,), jnp.float32),
            pltpu.VMEM((16,), jnp.int32),
            pltpu.VMEM_SHARED((16, 16), jnp.int32),
            pltpu.VMEM_SHARED((16, _BUF), jnp.int32),
            pltpu.VMEM_SHARED((16, _BUF), jnp.float32),
            pltpu.VMEM((16, 16), jnp.int32),
            pltpu.VMEM((16,), jnp.int32),
            pltpu.VMEM((16, _BUF), jnp.int32),
            pltpu.VMEM((16, _BUF), jnp.float32),
            pltpu.VMEM((16,), jnp.int32),
            pltpu.VMEM((16,), jnp.int32),
            pltpu.VMEM((16, 16), jnp.float32),
            pltpu.VMEM((16, 16), jnp.int32),
            pltpu.VMEM((96,), jnp.float32),
            pltpu.SemaphoreType.DMA,
            pltpu.SemaphoreType.DMA,
        ],
    )(scores, labels.astype(jnp.int32).reshape(_N // 16, 16),
      boxes.reshape(_N // 4, 16))
    return flat.reshape(_OUT_ROWS, 6)[:300][None]
